# baseline (speedup 1.0000x reference)
import functools

import jax
import jax.numpy as jnp
from jax import lax
from jax.experimental import pallas as pl
from jax.experimental.pallas import tpu as pltpu
from jax.experimental.pallas import tpu_sc as plsc

PAD = 0
B, H, D = 4096, 200, 32
BH = B * H
L = 16
NG = 13
ROWS_PER_CHUNK = 8
CH = ROWS_PER_CHUNK * H
CHP = CH + L


def _fused_sc(idx_pad, table, w):
    info = plsc.get_sparse_core_info()
    nc = info.num_cores
    nw = nc * info.num_subcores
    rows_per_w = B // nw
    n_chunks = rows_per_w // ROWS_PER_CHUNK
    mesh = plsc.VectorSubcoreMesh(core_axis_name="c", subcore_axis_name="s")

    @functools.partial(
        pl.kernel,
        mesh=mesh,
        out_type=jax.ShapeDtypeStruct((B, D), jnp.float32),
        scratch_types=[
            pltpu.VMEM((CHP,), jnp.int32),
            pltpu.VMEM((CHP, D), jnp.float32),
            pltpu.VMEM((D,), jnp.float32),
            pltpu.VMEM((ROWS_PER_CHUNK, D), jnp.float32),
            pltpu.SemaphoreType.DMA,
        ],
        compiler_params=pltpu.CompilerParams(use_tc_tiling_on_sc=False),
    )
    def k(idx_hbm, table_hbm, w_hbm, out_hbm, idx_v, rows_v, w_v, stage_v,
          sem):
        wid = lax.axis_index("s") * nc + lax.axis_index("c")
        row_base = wid * rows_per_w
        pltpu.sync_copy(w_hbm, w_v)
        lane = lax.iota(jnp.int32, L)
        w_lo = w_v[pl.ds(0, L)]
        w_hi = w_v[pl.ds(L, L)]
        perms = [lane ^ k for k in (1, 2, 4, 8)]
        _dnums = lax.GatherDimensionNumbers(
            offset_dims=(), collapsed_slice_dims=(0,), start_index_map=(0,))

        def _perm(u, p):
            return lax.gather(u, p[:, None], _dnums, (1,),
                              mode=lax.GatherScatterMode.PROMISE_IN_BOUNDS)

        def _allsum(u):
            for p in perms:
                u = u + _perm(u, p)
            return u

        def _allmax(u):
            for p in perms:
                u = jnp.maximum(u, _perm(u, p))
            return u

        def do_chunk(c, carry):
            row0 = row_base + c * ROWS_PER_CHUNK
            fbase = row0 * H
            pltpu.sync_copy(idx_hbm.at[pl.ds(fbase, CHP)], idx_v)
            pltpu.async_copy(table_hbm.at[idx_v], rows_v, sem).wait()

            def do_row(r, rcarry):
                rb = r * H
                attn = []
                for g in range(NG):
                    ib = rb + g * L
                    sv = jnp.zeros((L,), jnp.float32)
                    for j in range(L):
                        i = ib + j
                        v_lo = rows_v[i, pl.ds(0, L)]
                        v_hi = rows_v[i, pl.ds(L, L)]
                        s = _allsum(v_lo * w_lo + v_hi * w_hi)
                        sv = jnp.where(lane == j, s, sv)
                    idxg = idx_v[pl.ds(ib, L)]
                    valid = idxg != PAD
                    if g == NG - 1:
                        valid = valid & (lane < H - (NG - 1) * L)
                    attn.append(jnp.where(valid, sv, jnp.float32(-1e30)))
                m = attn[0]
                for g in range(1, NG):
                    m = jnp.maximum(m, attn[g])
                mm = _allmax(m)
                attn = [jnp.exp(a - mm) for a in attn]
                z = attn[0]
                for g in range(1, NG):
                    z = z + attn[g]
                inv = jnp.float32(1.0) / _allsum(z)
                attn = [a * inv for a in attn]
                acc_lo = jnp.zeros((L,), jnp.float32)
                acc_hi = jnp.zeros((L,), jnp.float32)
                for g in range(NG):
                    ib = rb + g * L
                    a = attn[g]
                    jmax = L if g < NG - 1 else H - (NG - 1) * L
                    for j in range(jmax):
                        i = ib + j
                        aj = _perm(a, jnp.full((L,), j, jnp.int32))
                        acc_lo = acc_lo + rows_v[i, pl.ds(0, L)] * aj
                        acc_hi = acc_hi + rows_v[i, pl.ds(L, L)] * aj
                stage_v[r, pl.ds(0, L)] = acc_lo
                stage_v[r, pl.ds(L, L)] = acc_hi
                return rcarry

            lax.fori_loop(0, ROWS_PER_CHUNK, do_row, 0)
            pltpu.sync_copy(stage_v, out_hbm.at[pl.ds(row0, ROWS_PER_CHUNK)])
            return carry

        lax.fori_loop(0, n_chunks, do_chunk, 0)

    return k(idx_pad, table, w)


def kernel(idx_tensor, table, attn_weight, attn_bias):
    del attn_bias
    idx_pad = jnp.concatenate(
        [idx_tensor.reshape(-1), jnp.zeros((L,), jnp.int32)])
    return _fused_sc(idx_pad, table, attn_weight.reshape(D))
